# baseline (device time: 26815 ns/iter reference)
import jax
import jax.numpy as jnp
from jax import lax
from jax.experimental import pallas as pl
from jax.experimental.pallas import tpu as pltpu

BLK = 512


def kernel(x, dy, gamma):
    m, d = x.shape
    nblk = m // BLK

    def body(x_ref, dy_ref, out_ref, acc_ref, recv_ref, send_sem, recv_sem):
        step = pl.program_id(0)
        nsteps = pl.num_programs(0)

        xb = x_ref[...].astype(jnp.float32)
        dyb = dy_ref[...].astype(jnp.float32)
        mu = jnp.mean(xb, axis=-1, keepdims=True)
        xc = xb - mu
        var = jnp.mean(xc * xc, axis=-1, keepdims=True)
        rstd = lax.rsqrt(var + 1e-5)
        xhat = xc * rstd
        dgamma_blk = jnp.sum(dyb * xhat, axis=0, keepdims=True)
        dbeta_blk = jnp.sum(dyb, axis=0, keepdims=True)
        blk = jnp.concatenate([dgamma_blk, dbeta_blk], axis=0)

        @pl.when(step == 0)
        def _():
            acc_ref[...] = blk

        @pl.when(step != 0)
        def _():
            acc_ref[...] = acc_ref[...] + blk

        @pl.when(step == nsteps - 1)
        def _():
            my_x = lax.axis_index("x")
            my_y = lax.axis_index("y")
            my_z = lax.axis_index("z")
            peer = (1 - my_x, my_y, my_z)

            barrier_sem = pltpu.get_barrier_semaphore()
            pl.semaphore_signal(
                barrier_sem,
                inc=1,
                device_id=peer,
                device_id_type=pl.DeviceIdType.MESH,
            )
            pl.semaphore_wait(barrier_sem, 1)

            rdma = pltpu.make_async_remote_copy(
                src_ref=acc_ref,
                dst_ref=recv_ref,
                send_sem=send_sem,
                recv_sem=recv_sem,
                device_id=peer,
                device_id_type=pl.DeviceIdType.MESH,
            )
            rdma.start()
            rdma.wait()
            out_ref[...] = acc_ref[...] + recv_ref[...]

    return pl.pallas_call(
        body,
        grid=(nblk,),
        in_specs=[
            pl.BlockSpec((BLK, d), lambda i: (i, 0)),
            pl.BlockSpec((BLK, d), lambda i: (i, 0)),
        ],
        out_specs=pl.BlockSpec((2, d), lambda i: (0, 0)),
        out_shape=jax.ShapeDtypeStruct((2, d), jnp.float32),
        scratch_shapes=[
            pltpu.VMEM((2, d), jnp.float32),
            pltpu.VMEM((2, d), jnp.float32),
            pltpu.SemaphoreType.DMA,
            pltpu.SemaphoreType.DMA,
        ],
        compiler_params=pltpu.CompilerParams(
            collective_id=0,
            dimension_semantics=("arbitrary",),
        ),
    )(x, dy)
